# baseline (speedup 1.0000x reference)
import functools

import jax
import jax.numpy as jnp
from jax import lax
from jax.experimental import pallas as pl
from jax.experimental.pallas import tpu as pltpu
from jax.experimental.pallas import tpu_sc as plsc

N_T = 26
VOCAB = 100000
D = 32
BATCH = 4096
L = 20

BAGS = N_T * BATCH
NW = 32
G = 64
IDX_PER_CHUNK = G * L
GW = 128
K = IDX_PER_CHUNK // GW

GROUPS = (7, 7, 6, 6)

VB = 25600
VB4 = VB // 4
NJ = -(-VOCAB // VB)
TV = NJ * VB


def _pooled_sc(idx3d, flat_tab, nbags):
    bags_per_w = nbags // NW
    chunks = bags_per_w // G
    mesh = plsc.VectorSubcoreMesh(core_axis_name="c", subcore_axis_name="s")

    @functools.partial(
        pl.kernel,
        out_type=jax.ShapeDtypeStruct((nbags, D), jnp.float32),
        mesh=mesh,
        scratch_types=[
            pltpu.VMEM((2, K, GW), jnp.int32),
            pltpu.VMEM((2, IDX_PER_CHUNK, D), jnp.float32),
            pltpu.VMEM((2, G, D), jnp.float32),
            pltpu.SemaphoreType.DMA,
            pltpu.SemaphoreType.DMA,
            pltpu.SemaphoreType.DMA,
        ],
        compiler_params=pltpu.CompilerParams(use_tc_tiling_on_sc=False),
    )
    def k(idx_hbm, tab_hbm, out_hbm, idx_v, rows_v, out_v, isem, gsem0, gsem1):
        wid = lax.axis_index("s") * 2 + lax.axis_index("c")
        bag_base = wid * bags_per_w
        gsems = (gsem0, gsem1)

        def fetch_idx(c, b):
            pltpu.async_copy(idx_hbm.at[wid * chunks + c], idx_v.at[b],
                             isem).wait()

        def fire_gathers(b):
            for j in range(K):
                pltpu.async_copy(tab_hbm.at[idx_v.at[b].at[j]],
                                 rows_v.at[b].at[pl.ds(j * GW, GW)], gsems[b])

        def wait_gathers(b):
            for j in range(K):
                pltpu.make_async_copy(tab_hbm.at[idx_v.at[b].at[j]],
                                      rows_v.at[b].at[pl.ds(j * GW, GW)],
                                      gsems[b]).wait()

        def pool_and_store(c, b):
            @pl.loop(0, G)
            def _(g):
                r0 = g * L
                a0 = rows_v[b, r0, pl.ds(0, 16)]
                a1 = rows_v[b, r0, pl.ds(16, 16)]
                for step in range(1, L):
                    a0 = a0 + rows_v[b, r0 + step, pl.ds(0, 16)]
                    a1 = a1 + rows_v[b, r0 + step, pl.ds(16, 16)]
                out_v[b, g, pl.ds(0, 16)] = a0
                out_v[b, g, pl.ds(16, 16)] = a1

            pltpu.sync_copy(out_v.at[b], out_hbm.at[pl.ds(bag_base + c * G, G)])

        fetch_idx(0, 0)
        fire_gathers(0)

        @pl.loop(0, chunks, step=2)
        def _(c):
            fetch_idx(c + 1, 1)
            fire_gathers(1)
            wait_gathers(0)
            pool_and_store(c, 0)

            @pl.when(c + 2 < chunks)
            def _():
                fetch_idx(c + 2, 0)
                fire_gathers(0)

            wait_gathers(1)
            pool_and_store(c + 1, 1)

    return k(idx3d, flat_tab)


def _detile_tc(tab_t, nt):
    def body(x_ref, o_ref):
        x = x_ref[0]
        y = jnp.transpose(x, (1, 0))
        for c in range(4):
            o_ref[0, :, c * D:(c + 1) * D] = y[c * VB4:(c + 1) * VB4, :]

    return pl.pallas_call(
        body,
        grid=(nt, NJ),
        in_specs=[pl.BlockSpec((1, D, VB), lambda t, j: (t, 0, j))],
        out_specs=pl.BlockSpec((1, VB4, 4 * D), lambda t, j: (t, j, 0)),
        out_shape=jax.ShapeDtypeStruct((nt, TV // 4, 4 * D), jnp.float32),
    )(tab_t)


BLK = 2048


def _mlp_tc(x128, w1, c1, w2, c2, w3, c3):
    pr = x128.shape[0]

    def body(x_ref, w1_ref, c1_ref, w2_ref, c2_ref, w3_ref, c3_ref, o_ref):
        dn = (((1,), (0,)), ((), ()))
        h = x_ref[...]
        h = lax.dot_general(h, w1_ref[...], dn) + c1_ref[...]
        h = lax.dot_general(h, w2_ref[...], dn) + c2_ref[...]
        h = lax.dot_general(h, w3_ref[...], dn) + c3_ref[...]
        o_ref[...] = h

    wspec = pl.BlockSpec((4 * D, 4 * D), lambda i: (0, 0))
    bspec = pl.BlockSpec((1, 4 * D), lambda i: (0, 0))
    return pl.pallas_call(
        body,
        grid=(pr // BLK,),
        in_specs=[pl.BlockSpec((BLK, 4 * D), lambda i: (i, 0)),
                  wspec, bspec, wspec, bspec, wspec, bspec],
        out_specs=pl.BlockSpec((BLK, 4 * D), lambda i: (i, 0)),
        out_shape=jax.ShapeDtypeStruct((pr, 4 * D), jnp.float32),
    )(x128, w1, c1, w2, c2, w3, c3)


def kernel(indices, tables, W1, b1, W2, b2, W3, b3):
    tab_t = jnp.transpose(tables, (0, 2, 1))
    pooled_parts = []
    t0 = 0
    for nt in GROUPS:
        v = indices[t0:t0 + nt].astype(jnp.int32)
        offs = (jnp.arange(nt, dtype=jnp.int32) * TV)[:, None, None]
        gidx = offs + (v // VB) * VB + 4 * (v % VB4) + (v % VB) // VB4
        nbags = nt * BATCH
        idx3d = gidx.reshape(nbags * L // (K * GW), K, GW)
        flat = _detile_tc(tab_t[t0:t0 + nt], nt).reshape(nt * TV, D)
        pooled_parts.append(_pooled_sc(idx3d, flat, nbags))
        t0 += nt
    pooled = jnp.concatenate(pooled_parts, axis=0)
    eye4 = jnp.eye(4, dtype=jnp.float32)
    out128 = _mlp_tc(pooled.reshape(BAGS // 4, 4 * D),
                     jnp.kron(eye4, W1.T), jnp.tile(b1, 4).reshape(1, 4 * D),
                     jnp.kron(eye4, W2.T), jnp.tile(b2, 4).reshape(1, 4 * D),
                     jnp.kron(eye4, W3.T), jnp.tile(b3, 4).reshape(1, 4 * D))
    return out128.reshape(BAGS, D)


# speedup vs baseline: 1.2973x; 1.2973x over previous
import functools

import jax
import jax.numpy as jnp
from jax import lax
from jax.experimental import pallas as pl
from jax.experimental.pallas import tpu as pltpu
from jax.experimental.pallas import tpu_sc as plsc

N_T = 26
VOCAB = 100000
D = 32
BATCH = 4096
L = 20

BAGS = N_T * BATCH
NW = 32
G = 64
IDX_PER_CHUNK = G * L
GW = 128
K = IDX_PER_CHUNK // GW

GROUPS = (7, 7, 6, 6)

VB = 25600
VB4 = VB // 4
NJ = -(-VOCAB // VB)
TV = NJ * VB


def _pooled_sc(idx3d, flat_tab, nbags):
    bags_per_w = nbags // NW
    chunks = bags_per_w // G
    mesh = plsc.VectorSubcoreMesh(core_axis_name="c", subcore_axis_name="s")

    @functools.partial(
        pl.kernel,
        out_type=jax.ShapeDtypeStruct((nbags, D), jnp.float32),
        mesh=mesh,
        scratch_types=[
            pltpu.VMEM((2, K, GW), jnp.int32),
            pltpu.VMEM((2, IDX_PER_CHUNK, D), jnp.float32),
            pltpu.VMEM((2, G, D), jnp.float32),
            pltpu.SemaphoreType.DMA,
            pltpu.SemaphoreType.DMA,
            pltpu.SemaphoreType.DMA,
        ],
        compiler_params=pltpu.CompilerParams(use_tc_tiling_on_sc=False),
    )
    def k(idx_hbm, tab_hbm, out_hbm, idx_v, rows_v, out_v, isem, gsem0, gsem1):
        wid = lax.axis_index("s") * 2 + lax.axis_index("c")
        bag_base = wid * bags_per_w
        gsems = (gsem0, gsem1)

        def fetch_idx(c, b):
            pltpu.async_copy(idx_hbm.at[wid * chunks + c], idx_v.at[b],
                             isem).wait()

        def fire_gathers(b):
            for j in range(K):
                pltpu.async_copy(tab_hbm.at[idx_v.at[b].at[j]],
                                 rows_v.at[b].at[pl.ds(j * GW, GW)], gsems[b])

        def wait_gathers(b):
            for j in range(K):
                pltpu.make_async_copy(tab_hbm.at[idx_v.at[b].at[j]],
                                      rows_v.at[b].at[pl.ds(j * GW, GW)],
                                      gsems[b]).wait()

        def pool_and_store(c, b):
            @pl.loop(0, G)
            def _(g):
                r0 = g * L
                a0 = rows_v[b, r0, pl.ds(0, 16)]
                a1 = rows_v[b, r0, pl.ds(16, 16)]
                for step in range(1, L):
                    a0 = a0 + rows_v[b, r0 + step, pl.ds(0, 16)]
                    a1 = a1 + rows_v[b, r0 + step, pl.ds(16, 16)]
                out_v[b, g, pl.ds(0, 16)] = a0
                out_v[b, g, pl.ds(16, 16)] = a1

            pltpu.sync_copy(out_v.at[b], out_hbm.at[pl.ds(bag_base + c * G, G)])

        fetch_idx(0, 0)
        fire_gathers(0)

        @pl.loop(0, chunks, step=2)
        def _(c):
            fetch_idx(c + 1, 1)
            fire_gathers(1)
            wait_gathers(0)
            pool_and_store(c, 0)

            @pl.when(c + 2 < chunks)
            def _():
                fetch_idx(c + 2, 0)
                fire_gathers(0)

            wait_gathers(1)
            pool_and_store(c + 1, 1)

    return k(idx3d, flat_tab)


def _detile_tc(tab_t, nt):
    def body(x_ref, o_ref):
        x = x_ref[0]
        y = jnp.transpose(x, (1, 0))
        for c in range(4):
            o_ref[0, :, c * D:(c + 1) * D] = y[c * VB4:(c + 1) * VB4, :]

    return pl.pallas_call(
        body,
        grid=(nt, NJ),
        in_specs=[pl.BlockSpec((1, D, VB), lambda t, j: (t, 0, j))],
        out_specs=pl.BlockSpec((1, VB4, 4 * D), lambda t, j: (t, j, 0)),
        out_shape=jax.ShapeDtypeStruct((nt, TV // 4, 4 * D), jnp.float32),
    )(tab_t)


BLK = 2048


def _mlp_tc(x128, w1, c1, w2, c2, w3, c3):
    pr = x128.shape[0]

    def body(x_ref, w1_ref, c1_ref, w2_ref, c2_ref, w3_ref, c3_ref, o_ref):
        dn = (((1,), (0,)), ((), ()))
        h = x_ref[...]
        h = lax.dot_general(h, w1_ref[...], dn) + c1_ref[...]
        h = lax.dot_general(h, w2_ref[...], dn) + c2_ref[...]
        h = lax.dot_general(h, w3_ref[...], dn) + c3_ref[...]
        o_ref[...] = h

    wspec = pl.BlockSpec((4 * D, 4 * D), lambda i: (0, 0))
    bspec = pl.BlockSpec((1, 4 * D), lambda i: (0, 0))
    return pl.pallas_call(
        body,
        grid=(pr // BLK,),
        in_specs=[pl.BlockSpec((BLK, 4 * D), lambda i: (i, 0)),
                  wspec, bspec, wspec, bspec, wspec, bspec],
        out_specs=pl.BlockSpec((BLK, 4 * D), lambda i: (i, 0)),
        out_shape=jax.ShapeDtypeStruct((pr, 4 * D), jnp.float32),
    )(x128, w1, c1, w2, c2, w3, c3)


def kernel(indices, tables, W1, b1, W2, b2, W3, b3):
    tab_t = jnp.transpose(tables, (0, 2, 1))
    v = indices.astype(jnp.int32)
    offs = (jnp.arange(N_T, dtype=jnp.int32) * TV)[:, None, None]
    gidx = offs + (v // VB) * VB + 4 * (v % VB4) + (v % VB) // VB4
    idx3d = gidx.reshape(BAGS * L // (K * GW), K, GW)
    flat = _detile_tc(tab_t, N_T).reshape(N_T * TV, D)
    pooled = _pooled_sc(idx3d, flat, BAGS)
    eye4 = jnp.eye(4, dtype=jnp.float32)
    out128 = _mlp_tc(pooled.reshape(BAGS // 4, 4 * D),
                     jnp.kron(eye4, W1.T), jnp.tile(b1, 4).reshape(1, 4 * D),
                     jnp.kron(eye4, W2.T), jnp.tile(b2, 4).reshape(1, 4 * D),
                     jnp.kron(eye4, W3.T), jnp.tile(b3, 4).reshape(1, 4 * D))
    return out128.reshape(BAGS, D)


# speedup vs baseline: 1.3062x; 1.0068x over previous
import functools

import jax
import jax.numpy as jnp
from jax import lax
from jax.experimental import pallas as pl
from jax.experimental.pallas import tpu as pltpu
from jax.experimental.pallas import tpu_sc as plsc

N_T = 26
VOCAB = 100000
D = 32
BATCH = 4096
L = 20

BAGS = N_T * BATCH
NW = 32
G = 64
IDX_PER_CHUNK = G * L
GW = 128
K = IDX_PER_CHUNK // GW

GROUPS = (7, 7, 6, 6)

VB = 25600
VB4 = VB // 4
NJ = -(-VOCAB // VB)
TV = NJ * VB


def _pooled_sc(idx3d, flat_tab, nbags):
    bags_per_w = nbags // NW
    chunks = bags_per_w // G
    mesh = plsc.VectorSubcoreMesh(core_axis_name="c", subcore_axis_name="s")

    @functools.partial(
        pl.kernel,
        out_type=jax.ShapeDtypeStruct((nbags, D), jnp.float32),
        mesh=mesh,
        scratch_types=[
            pltpu.VMEM((2, K, GW), jnp.int32),
            pltpu.VMEM((2, IDX_PER_CHUNK, D), jnp.float32),
            pltpu.VMEM((2, G, D), jnp.float32),
            pltpu.SemaphoreType.DMA,
            pltpu.SemaphoreType.DMA,
            pltpu.SemaphoreType.DMA,
        ],
        compiler_params=pltpu.CompilerParams(use_tc_tiling_on_sc=False),
    )
    def k(idx_hbm, tab_hbm, out_hbm, idx_v, rows_v, out_v, isem, gsem0, gsem1):
        wid = lax.axis_index("s") * 2 + lax.axis_index("c")
        bag_base = wid * bags_per_w
        gsems = (gsem0, gsem1)

        def fetch_idx(c, b):
            pltpu.async_copy(idx_hbm.at[wid * chunks + c], idx_v.at[b],
                             isem).wait()

        def fire_gathers(b):
            for j in range(K):
                pltpu.async_copy(tab_hbm.at[idx_v.at[b].at[j]],
                                 rows_v.at[b].at[pl.ds(j * GW, GW)], gsems[b])

        def wait_gathers(b):
            for j in range(K):
                pltpu.make_async_copy(tab_hbm.at[idx_v.at[b].at[j]],
                                      rows_v.at[b].at[pl.ds(j * GW, GW)],
                                      gsems[b]).wait()

        def pool_and_store(c, b):
            @pl.loop(0, G, step=2)
            def _(g):
                r0 = g * L
                r1 = r0 + L
                a0 = rows_v[b, r0, pl.ds(0, 16)]
                a1 = rows_v[b, r0, pl.ds(16, 16)]
                a2 = rows_v[b, r1, pl.ds(0, 16)]
                a3 = rows_v[b, r1, pl.ds(16, 16)]
                for step in range(1, L):
                    a0 = a0 + rows_v[b, r0 + step, pl.ds(0, 16)]
                    a1 = a1 + rows_v[b, r0 + step, pl.ds(16, 16)]
                    a2 = a2 + rows_v[b, r1 + step, pl.ds(0, 16)]
                    a3 = a3 + rows_v[b, r1 + step, pl.ds(16, 16)]
                out_v[b, g, pl.ds(0, 16)] = a0
                out_v[b, g, pl.ds(16, 16)] = a1
                out_v[b, g + 1, pl.ds(0, 16)] = a2
                out_v[b, g + 1, pl.ds(16, 16)] = a3

            pltpu.sync_copy(out_v.at[b], out_hbm.at[pl.ds(bag_base + c * G, G)])

        fetch_idx(0, 0)
        fire_gathers(0)

        @pl.loop(0, chunks, step=2)
        def _(c):
            fetch_idx(c + 1, 1)
            fire_gathers(1)
            wait_gathers(0)
            pool_and_store(c, 0)

            @pl.when(c + 2 < chunks)
            def _():
                fetch_idx(c + 2, 0)
                fire_gathers(0)

            wait_gathers(1)
            pool_and_store(c + 1, 1)

    return k(idx3d, flat_tab)


def _detile_tc(tab_t, nt):
    def body(x_ref, o_ref):
        x = x_ref[0]
        y = jnp.transpose(x, (1, 0))
        for c in range(4):
            o_ref[0, :, c * D:(c + 1) * D] = y[c * VB4:(c + 1) * VB4, :]

    return pl.pallas_call(
        body,
        grid=(nt, NJ),
        in_specs=[pl.BlockSpec((1, D, VB), lambda t, j: (t, 0, j))],
        out_specs=pl.BlockSpec((1, VB4, 4 * D), lambda t, j: (t, j, 0)),
        out_shape=jax.ShapeDtypeStruct((nt, TV // 4, 4 * D), jnp.float32),
    )(tab_t)


BLK = 2048


def _mlp_tc(x128, w1, c1, w2, c2, w3, c3):
    pr = x128.shape[0]

    def body(x_ref, w1_ref, c1_ref, w2_ref, c2_ref, w3_ref, c3_ref, o_ref):
        dn = (((1,), (0,)), ((), ()))
        h = x_ref[...]
        h = lax.dot_general(h, w1_ref[...], dn) + c1_ref[...]
        h = lax.dot_general(h, w2_ref[...], dn) + c2_ref[...]
        h = lax.dot_general(h, w3_ref[...], dn) + c3_ref[...]
        o_ref[...] = h

    wspec = pl.BlockSpec((4 * D, 4 * D), lambda i: (0, 0))
    bspec = pl.BlockSpec((1, 4 * D), lambda i: (0, 0))
    return pl.pallas_call(
        body,
        grid=(pr // BLK,),
        in_specs=[pl.BlockSpec((BLK, 4 * D), lambda i: (i, 0)),
                  wspec, bspec, wspec, bspec, wspec, bspec],
        out_specs=pl.BlockSpec((BLK, 4 * D), lambda i: (i, 0)),
        out_shape=jax.ShapeDtypeStruct((pr, 4 * D), jnp.float32),
    )(x128, w1, c1, w2, c2, w3, c3)


def kernel(indices, tables, W1, b1, W2, b2, W3, b3):
    tab_t = jnp.transpose(tables, (0, 2, 1))
    v = indices.astype(jnp.int32)
    offs = (jnp.arange(N_T, dtype=jnp.int32) * TV)[:, None, None]
    gidx = offs + (v // VB) * VB + 4 * (v % VB4) + (v % VB) // VB4
    idx3d = gidx.reshape(BAGS * L // (K * GW), K, GW)
    flat = _detile_tc(tab_t, N_T).reshape(N_T * TV, D)
    pooled = _pooled_sc(idx3d, flat, BAGS)
    eye4 = jnp.eye(4, dtype=jnp.float32)
    out128 = _mlp_tc(pooled.reshape(BAGS // 4, 4 * D),
                     jnp.kron(eye4, W1.T), jnp.tile(b1, 4).reshape(1, 4 * D),
                     jnp.kron(eye4, W2.T), jnp.tile(b2, 4).reshape(1, 4 * D),
                     jnp.kron(eye4, W3.T), jnp.tile(b3, 4).reshape(1, 4 * D))
    return out128.reshape(BAGS, D)
